# baseline (speedup 1.0000x reference)
import functools

import jax
import jax.numpy as jnp
from jax import lax
from jax.experimental import pallas as pl
from jax.experimental.pallas import tpu as pltpu
from jax.experimental.pallas import tpu_sc as plsc

_NB = 16
_BS = 64
_HID = 128
_NACT = 16
_FEAT = 3
_L = 16
_GPT = 2


@functools.partial(
    pl.kernel,
    out_type=jax.ShapeDtypeStruct((_BS // _GPT, _L), jnp.float32),
    mesh=plsc.VectorSubcoreMesh(core_axis_name="c", subcore_axis_name="s"),
    scratch_types=[
        pltpu.VMEM((_GPT * _FEAT * _L,), jnp.float32),
        pltpu.VMEM((_GPT * _NACT,), jnp.float32),
        pltpu.VMEM((_FEAT * _HID,), jnp.float32),
        pltpu.VMEM((_HID,), jnp.float32),
        pltpu.VMEM((_HID * _HID,), jnp.float32),
        pltpu.VMEM((_HID,), jnp.float32),
        pltpu.VMEM((_HID * _HID,), jnp.float32),
        pltpu.VMEM((_HID,), jnp.float32),
        pltpu.VMEM((_HID * _NACT,), jnp.float32),
        pltpu.VMEM((_NACT,), jnp.float32),
        pltpu.VMEM((_GPT * _HID,), jnp.float32),
        pltpu.VMEM((_GPT * _HID,), jnp.float32),
        pltpu.VMEM((_GPT * _HID,), jnp.float32),
        pltpu.VMEM((_L,), jnp.float32),
        pltpu.SemaphoreType.DMA,
    ],
)
def _sc_full(ut_hbm, act_hbm, wet_hbm, be_hbm, wgt_hbm, bg_hbm, w1t_hbm,
             b1_hbm, w2t_hbm, b2_hbm, out_hbm,
             ut_v, act_v, wet_v, be_v, wgt_v, bg_v, w1t_v, b1_v, w2t_v,
             b2_v, xm_v, g_v, h_v, out_v, sem):
    wid = lax.axis_index("s") * 2 + lax.axis_index("c")
    copies = [
        pltpu.async_copy(ut_hbm.at[pl.ds(wid * (_GPT * _FEAT * _L),
                                         _GPT * _FEAT * _L)], ut_v, sem),
        pltpu.async_copy(act_hbm.at[pl.ds(wid * (_GPT * _NACT),
                                          _GPT * _NACT)], act_v, sem),
        pltpu.async_copy(wet_hbm, wet_v, sem),
        pltpu.async_copy(be_hbm, be_v, sem),
        pltpu.async_copy(wgt_hbm, wgt_v, sem),
        pltpu.async_copy(bg_hbm, bg_v, sem),
        pltpu.async_copy(w1t_hbm, w1t_v, sem),
        pltpu.async_copy(b1_hbm, b1_v, sem),
        pltpu.async_copy(w2t_hbm, w2t_v, sem),
        pltpu.async_copy(b2_hbm, b2_v, sem),
    ]
    for c in copies:
        c.wait()

    iota = lax.iota(jnp.int32, _L)
    idx_const = [iota * 0 + l for l in range(_L)]

    def bfly(v, op):
        for s in (8, 4, 2, 1):
            v = op(v, v.at[jnp.bitwise_xor(iota, s)].get(
                mode="promise_in_bounds"))
        return v

    for gi in range(_GPT):
        us = [bfly(ut_v[pl.ds(gi * _FEAT * _L + k * _L, _L)], jnp.add)
              * (1.0 / _NB) for k in range(_FEAT)]
        for j in range(_HID // _L):
            xm = be_v[pl.ds(_L * j, _L)]
            for k in range(_FEAT):
                xm = xm + us[k] * wet_v[pl.ds(k * _HID + _L * j, _L)]
            xm_v[pl.ds(gi * _HID + _L * j, _L)] = xm

    def mm_stage(src_v, w_v, b_v, ncol):
        nj = ncol // _L
        accs = [b_v[pl.ds(_L * j, _L)] for j in range(nj)] * _GPT

        def body(kg, carry):
            flat = list(carry)
            xk = [src_v[pl.ds(gi * _HID + kg * _L, _L)] for gi in range(_GPT)]
            for l in range(_L):
                k = kg * _L + l
                sp = [xk[gi].at[idx_const[l]].get(mode="promise_in_bounds")
                      for gi in range(_GPT)]
                for j in range(nj):
                    w = w_v[pl.ds(k * ncol + _L * j, _L)]
                    for gi in range(_GPT):
                        flat[gi * nj + j] = flat[gi * nj + j] + sp[gi] * w
            return tuple(flat)

        flat = lax.fori_loop(0, _HID // _L, body,
                             tuple(accs[:nj]) + tuple(accs[nj:]))
        return [list(flat[gi * nj:(gi + 1) * nj]) for gi in range(_GPT)]

    g = mm_stage(xm_v, wgt_v, bg_v, _HID)
    for gi in range(_GPT):
        for j in range(_HID // _L):
            g_v[pl.ds(gi * _HID + _L * j, _L)] = g[gi][j]

    h = mm_stage(g_v, w1t_v, b1_v, _HID)
    for gi in range(_GPT):
        for j in range(_HID // _L):
            hv = h[gi][j]
            h_v[pl.ds(gi * _HID + _L * j, _L)] = jnp.where(
                hv >= 0, hv, 0.01 * hv)

    qa = mm_stage(h_v, w2t_v, b2_v, _NACT)

    acc_out = jnp.zeros((_L,), jnp.float32)
    for gi in range(_GPT):
        av = act_v[pl.ds(gi * _NACT, _NACT)]
        m_sp = bfly(av, jnp.maximum)
        idx_sp = bfly(jnp.where(av == m_sp, iota, _NACT), jnp.minimum)
        q_sp = qa[gi][0].at[idx_sp].get(mode="promise_in_bounds")
        acc_out = jnp.where(iota == gi, q_sp, acc_out)
    out_v[...] = acc_out
    pltpu.sync_copy(out_v, out_hbm.at[wid])


def kernel(inps, unary_tensor, W_emb, b_emb, W_gcn, b_gcn, W1, b1, W2, b2):
    actions = inps[0, 1].reshape(-1)
    ut = unary_tensor.transpose(0, 2, 1).reshape(-1)
    out = _sc_full(ut, actions, W_emb.T.reshape(-1), b_emb,
                   W_gcn.T.reshape(-1), b_gcn, W1.T.reshape(-1), b1,
                   W2.T.reshape(-1), b2)
    return out[:, :_GPT].reshape(_BS, 1)


# speedup vs baseline: 1.3579x; 1.3579x over previous
import functools

import jax
import jax.numpy as jnp
from jax import lax
from jax.experimental import pallas as pl
from jax.experimental.pallas import tpu as pltpu
from jax.experimental.pallas import tpu_sc as plsc

_NB = 16
_BS = 64
_HID = 128
_NACT = 16
_FEAT = 3
_L = 16
_RPT = 2


def _dense_kernel(x_ref, wt_ref, be_ref, wg_ref, bg_ref, w1_ref, b1_ref,
                  w2_ref, b2_ref, out_ref):
    xm = jnp.dot(x_ref[...], wt_ref[...],
                 preferred_element_type=jnp.float32) * (1.0 / _NB) + be_ref[...]
    g = jnp.dot(xm, wg_ref[...], preferred_element_type=jnp.float32) + bg_ref[...]
    h = jnp.dot(g, w1_ref[...], preferred_element_type=jnp.float32) + b1_ref[...]
    h = jnp.where(h >= 0, h, 0.01 * h)
    out_ref[...] = jnp.dot(h, w2_ref[...],
                           preferred_element_type=jnp.float32) + b2_ref[...]


def _select_kernel(q_ref, m_ref, out_ref):
    out_ref[...] = jnp.sum(q_ref[...] * m_ref[...], axis=1, keepdims=True)


@functools.partial(
    pl.kernel,
    out_type=jax.ShapeDtypeStruct((_BS * _NACT,), jnp.float32),
    mesh=plsc.VectorSubcoreMesh(core_axis_name="c", subcore_axis_name="s"),
    scratch_types=[
        pltpu.VMEM((_RPT * _NACT,), jnp.float32),
        pltpu.VMEM((_RPT * _NACT,), jnp.float32),
    ],
)
def _sc_onehot(act_hbm, out_hbm, act_v, out_v):
    wid = lax.axis_index("s") * 2 + lax.axis_index("c")
    base = wid * (_RPT * _NACT)
    pltpu.sync_copy(act_hbm.at[pl.ds(base, _RPT * _NACT)], act_v)
    iota = lax.iota(jnp.int32, _L)

    def bfly(v, op):
        for s in (8, 4, 2, 1):
            v = op(v, v.at[jnp.bitwise_xor(iota, s)].get(
                mode="promise_in_bounds"))
        return v

    for r in range(_RPT):
        av = act_v[pl.ds(r * _NACT, _NACT)]
        m_sp = bfly(av, jnp.maximum)
        idx_sp = bfly(jnp.where(av == m_sp, iota, _NACT), jnp.minimum)
        out_v[pl.ds(r * _NACT, _NACT)] = jnp.where(
            iota == idx_sp, 1.0, 0.0)
    pltpu.sync_copy(out_v, out_hbm.at[pl.ds(base, _RPT * _NACT)])


def kernel(inps, unary_tensor, W_emb, b_emb, W_gcn, b_gcn, W1, b1, W2, b2):
    actions = inps[0, 1]
    mask = _sc_onehot(actions.reshape(-1)).reshape(_BS, _NACT)
    x = unary_tensor.reshape(_BS, _NB * _FEAT)
    wt = jnp.tile(W_emb.T, (_NB, 1))
    all_q = pl.pallas_call(
        _dense_kernel,
        out_shape=jax.ShapeDtypeStruct((_BS, _NACT), jnp.float32),
    )(x, wt, b_emb.reshape(1, _HID), W_gcn.T, b_gcn.reshape(1, _HID),
      W1.T, b1.reshape(1, _HID), W2.T, b2.reshape(1, _NACT))
    return pl.pallas_call(
        _select_kernel,
        out_shape=jax.ShapeDtypeStruct((_BS, 1), jnp.float32),
    )(all_q, mask)


# speedup vs baseline: 1.3992x; 1.0304x over previous
import functools

import jax
import jax.numpy as jnp
from jax import lax
from jax.experimental import pallas as pl
from jax.experimental.pallas import tpu as pltpu
from jax.experimental.pallas import tpu_sc as plsc

_NB = 16
_BS = 64
_HID = 128
_NACT = 16
_FEAT = 3
_ROWS_PER_TILE = 8
_NTILES = _BS // _ROWS_PER_TILE


def _dense_kernel(x_ref, wt_ref, be_ref, wg_ref, bg_ref, w1_ref, b1_ref,
                  w2_ref, b2_ref, out_ref):
    xm = jnp.dot(x_ref[...], wt_ref[...],
                 preferred_element_type=jnp.float32) * (1.0 / _NB) + be_ref[...]
    g = jnp.dot(xm, wg_ref[...], preferred_element_type=jnp.float32) + bg_ref[...]
    h = jnp.dot(g, w1_ref[...], preferred_element_type=jnp.float32) + b1_ref[...]
    h = jnp.where(h >= 0, h, 0.01 * h)
    out_ref[...] = jnp.dot(h, w2_ref[...],
                           preferred_element_type=jnp.float32) + b2_ref[...]


@functools.partial(
    pl.kernel,
    out_type=jax.ShapeDtypeStruct((_BS,), jnp.float32),
    mesh=plsc.VectorSubcoreMesh(core_axis_name="c", subcore_axis_name="s"),
    scratch_types=[
        pltpu.VMEM((_ROWS_PER_TILE, _NACT), jnp.float32),
        pltpu.VMEM((_ROWS_PER_TILE, _NACT), jnp.float32),
        pltpu.VMEM((_NACT,), jnp.float32),
    ],
)
def _sc_select(act_hbm, q_hbm, out_hbm, act_v, q_v, out_v):
    wid = lax.axis_index("s") * 2 + lax.axis_index("c")

    @pl.when(wid < _NTILES)
    def _():
        base = wid * _ROWS_PER_TILE
        pltpu.sync_copy(act_hbm.at[pl.ds(base, _ROWS_PER_TILE), :], act_v)
        pltpu.sync_copy(q_hbm.at[pl.ds(base, _ROWS_PER_TILE), :], q_v)
        iota = lax.iota(jnp.int32, _NACT)

        def splat_reduce(v, op):
            for s in (8, 4, 2, 1):
                perm = jnp.bitwise_xor(iota, s)
                v = op(v, v.at[perm].get(mode="promise_in_bounds"))
            return v

        acc = jnp.zeros((_NACT,), jnp.float32)
        for r in range(_ROWS_PER_TILE):
            av = act_v[r, :]
            m_sp = splat_reduce(av, jnp.maximum)
            idx_sp = splat_reduce(jnp.where(av == m_sp, iota, _NACT),
                                  jnp.minimum)
            q_sp = q_v[r, :].at[idx_sp].get(mode="promise_in_bounds")
            acc = jnp.where(iota == r, q_sp, acc)
        out_v[...] = acc
        pltpu.sync_copy(out_v.at[pl.ds(0, _ROWS_PER_TILE)],
                        out_hbm.at[pl.ds(base, _ROWS_PER_TILE)])


def kernel(inps, unary_tensor, W_emb, b_emb, W_gcn, b_gcn, W1, b1, W2, b2):
    actions = inps[0, 1]
    x = unary_tensor.reshape(_BS, _NB * _FEAT)
    wt = jnp.tile(W_emb.T, (_NB, 1))
    all_q = pl.pallas_call(
        _dense_kernel,
        out_shape=jax.ShapeDtypeStruct((_BS, _NACT), jnp.float32),
    )(x, wt, b_emb.reshape(1, _HID), W_gcn.T, b_gcn.reshape(1, _HID),
      W1.T, b1.reshape(1, _HID), W2.T, b2.reshape(1, _NACT))
    return _sc_select(actions, all_q).reshape(_BS, 1)


# speedup vs baseline: 1.5230x; 1.0885x over previous
import functools

import jax
import jax.numpy as jnp
from jax import lax
from jax.experimental import pallas as pl
from jax.experimental.pallas import tpu as pltpu
from jax.experimental.pallas import tpu_sc as plsc

_NB = 16
_BS = 64
_HID = 128
_NACT = 16
_FEAT = 3
_ROWS_PER_TILE = 8
_NTILES = _BS // _ROWS_PER_TILE


def _dot_t(a, b):
    return lax.dot_general(a, b, (((1,), (1,)), ((), ())),
                           preferred_element_type=jnp.float32)


def _dense_kernel(x_ref, we_ref, be_ref, wg_ref, bg_ref, w1_ref, b1_ref,
                  w2_ref, b2_ref, out_ref):
    row = jax.lax.broadcasted_iota(jnp.int32, (_NB * _FEAT, _FEAT), 0)
    col = jax.lax.broadcasted_iota(jnp.int32, (_NB * _FEAT, _FEAT), 1)
    t = jnp.where(row % _FEAT == col, 1.0, 0.0)
    us = jnp.dot(x_ref[...], t, preferred_element_type=jnp.float32)
    xm = _dot_t(us, we_ref[...]) * (1.0 / _NB) + be_ref[...]
    g = _dot_t(xm, wg_ref[...]) + bg_ref[...]
    h = _dot_t(g, w1_ref[...]) + b1_ref[...]
    h = jnp.where(h >= 0, h, 0.01 * h)
    out_ref[...] = _dot_t(h, w2_ref[...]) + b2_ref[...]


@functools.partial(
    pl.kernel,
    out_type=jax.ShapeDtypeStruct((_BS,), jnp.float32),
    mesh=plsc.VectorSubcoreMesh(core_axis_name="c", subcore_axis_name="s"),
    scratch_types=[
        pltpu.VMEM((_ROWS_PER_TILE, _NACT), jnp.float32),
        pltpu.VMEM((_ROWS_PER_TILE, _NACT), jnp.float32),
        pltpu.VMEM((_NACT,), jnp.float32),
    ],
)
def _sc_select(act_hbm, q_hbm, out_hbm, act_v, q_v, out_v):
    wid = lax.axis_index("s") * 2 + lax.axis_index("c")

    @pl.when(wid < _NTILES)
    def _():
        base = wid * _ROWS_PER_TILE
        pltpu.sync_copy(act_hbm.at[pl.ds(base, _ROWS_PER_TILE), :], act_v)
        pltpu.sync_copy(q_hbm.at[pl.ds(base, _ROWS_PER_TILE), :], q_v)
        iota = lax.iota(jnp.int32, _NACT)

        def splat_reduce(v, op):
            for s in (8, 4, 2, 1):
                perm = jnp.bitwise_xor(iota, s)
                v = op(v, v.at[perm].get(mode="promise_in_bounds"))
            return v

        acc = jnp.zeros((_NACT,), jnp.float32)
        for r in range(_ROWS_PER_TILE):
            av = act_v[r, :]
            m_sp = splat_reduce(av, jnp.maximum)
            idx_sp = splat_reduce(jnp.where(av == m_sp, iota, _NACT),
                                  jnp.minimum)
            q_sp = q_v[r, :].at[idx_sp].get(mode="promise_in_bounds")
            acc = jnp.where(iota == r, q_sp, acc)
        out_v[...] = acc
        pltpu.sync_copy(out_v.at[pl.ds(0, _ROWS_PER_TILE)],
                        out_hbm.at[pl.ds(base, _ROWS_PER_TILE)])


def kernel(inps, unary_tensor, W_emb, b_emb, W_gcn, b_gcn, W1, b1, W2, b2):
    actions = inps[0, 1]
    x = unary_tensor.reshape(_BS, _NB * _FEAT)
    all_q = pl.pallas_call(
        _dense_kernel,
        out_shape=jax.ShapeDtypeStruct((_BS, _NACT), jnp.float32),
    )(x, W_emb, b_emb.reshape(1, _HID), W_gcn, b_gcn.reshape(1, _HID),
      W1, b1.reshape(1, _HID), W2, b2.reshape(1, _NACT))
    return _sc_select(actions, all_q).reshape(_BS, 1)
